# baseline (speedup 1.0000x reference)
import jax
import jax.numpy as jnp
from jax import lax
from jax.experimental import pallas as pl
from jax.experimental.pallas import tpu as pltpu
from jax.experimental.pallas import tpu_sc as plsc

VOCAB = 1000000
D = 64
SEQ = 4096
N_NODES = 50000
SPAN = 8

NC = 2
NS = 16
LANES = 16

DCHUNKS = D // LANES
NGROUPS = (NC * NS) // DCHUNKS
NODES_PER_GROUP = N_NODES // NGROUPS
CHUNK = 250
NCHUNKS = NODES_PER_GROUP // CHUNK
ROWS_PER_SUB = SEQ // NS


def _body(tok_hbm, span_hbm, table_hbm, out_hbm,
          tok_v, idx_v, gbuf, emb_v, span_v, out_v, emb_s, sem):
    core = lax.axis_index("c")
    sub = lax.axis_index("s")
    group = core * (NS // DCHUNKS) + sub // DCHUNKS
    cchunk = sub % DCHUNKS
    tok0 = sub * ROWS_PER_SUB

    pltpu.sync_copy(tok_hbm, tok_v)

    iota = lax.iota(jnp.int32, LANES)

    def idx_body(s, carry):
        tv = tok_v[pl.ds(tok0 + s * LANES, LANES)]
        for l in range(LANES):
            t = tv[l]
            for m in range(D // LANES):
                idx_v[s * LANES + l, pl.ds(m * LANES, LANES)] = (
                    (m * LANES + iota) * VOCAB + t)
        return carry

    lax.fori_loop(0, ROWS_PER_SUB // LANES, idx_body, 0)

    copies = []
    for i in range(ROWS_PER_SUB):
        copies.append(pltpu.async_copy(
            table_hbm.at[idx_v.at[i]],
            gbuf.at[i],
            sem))
    for cp in copies:
        cp.wait()

    for cc in range(DCHUNKS):
        pltpu.sync_copy(gbuf.at[:, pl.ds(cc * LANES, LANES)],
                        emb_s.at[cc, pl.ds(tok0, ROWS_PER_SUB)])
    plsc.subcore_barrier()

    pltpu.sync_copy(emb_s.at[cchunk], emb_v)

    def chunk_body(t, carry):
        node_base = group * NODES_PER_GROUP + t * CHUNK
        pltpu.sync_copy(span_hbm.at[pl.ds(node_base * SPAN, CHUNK * SPAN)],
                        span_v)

        def pair_body(k2, carry2):
            sv = span_v[pl.ds(k2 * 2 * SPAN, 2 * SPAN)]
            scale = jnp.float32(1.0 / SPAN)
            for half in range(2):
                rows = [emb_v[sv[half * SPAN + j]] for j in range(SPAN)]
                while len(rows) > 1:
                    rows = [rows[i] + rows[i + 1]
                            for i in range(0, len(rows), 2)]
                out_v[k2 * 2 + half] = rows[0] * scale
            return carry2

        lax.fori_loop(0, CHUNK // 2, pair_body, 0, unroll=5)
        pltpu.sync_copy(out_v,
                        out_hbm.at[pl.ds(node_base, CHUNK),
                                   pl.ds(cchunk * LANES, LANES)])
        return carry

    lax.fori_loop(0, NCHUNKS, chunk_body, 0)


@jax.jit
def _graph_embed(tok, span, table1d):
    mesh = plsc.VectorSubcoreMesh(core_axis_name="c", subcore_axis_name="s",
                                  num_cores=NC, num_subcores=NS)
    f = pl.kernel(
        _body,
        out_type=jax.ShapeDtypeStruct((N_NODES, D), jnp.float32),
        mesh=mesh,
        scratch_types=[
            pltpu.VMEM((SEQ,), jnp.int32),
            pltpu.VMEM((ROWS_PER_SUB, D), jnp.int32),
            pltpu.VMEM((ROWS_PER_SUB, D), jnp.float32),
            pltpu.VMEM((SEQ, LANES), jnp.float32),
            pltpu.VMEM((CHUNK * SPAN,), jnp.int32),
            pltpu.VMEM((CHUNK, LANES), jnp.float32),
            pltpu.VMEM_SHARED((DCHUNKS, SEQ, LANES),
                              jnp.float32),
            pltpu.SemaphoreType.DMA,
        ],
        compiler_params=pltpu.CompilerParams(use_tc_tiling_on_sc=False,
                                             needs_layout_passes=False),
    )
    return f(tok, span, table1d)


def kernel(token_ids, node_span_idx, table):
    tok = token_ids.reshape(-1).astype(jnp.int32)
    span = node_span_idx.reshape(-1).astype(jnp.int32)
    table1d = table.T.reshape(-1)
    return _graph_embed(tok, span, table1d)


# speedup vs baseline: 6.6392x; 6.6392x over previous
import jax
import jax.numpy as jnp
from jax import lax
from jax.experimental import pallas as pl
from jax.experimental.pallas import tpu as pltpu
from jax.experimental.pallas import tpu_sc as plsc

VOCAB = 1000000
D = 64
SEQ = 4096
N_NODES = 50000
SPAN = 8

NC = 2
NS = 16
LANES = 16

DCHUNKS = D // LANES
NGROUPS = (NC * NS) // DCHUNKS
NODES_PER_GROUP = N_NODES // NGROUPS
CHUNK = 250
NCHUNKS = NODES_PER_GROUP // CHUNK
ROWS_PER_SUB = SEQ // NS


def _body(tok_hbm, span_hbm, table_hbm, out_hbm,
          tok_v, gbuf, emb_v, span_v, out_v,
          emb_s, sem_g, sem_span, sem_out):
    core = lax.axis_index("c")
    sub = lax.axis_index("s")
    group = core * (NS // DCHUNKS) + sub // DCHUNKS
    cchunk = sub % DCHUNKS
    tok0 = sub * ROWS_PER_SUB
    node0 = group * NODES_PER_GROUP

    pltpu.sync_copy(tok_hbm, tok_v)
    fetches = []
    for s in range(ROWS_PER_SUB // LANES):
        tv = tok_v[pl.ds(tok0 + s * LANES, LANES)]
        for l in range(LANES):
            fetches.append(pltpu.async_copy(
                table_hbm.at[pl.ds(tv[l] * D, D)],
                gbuf.at[s * LANES + l],
                sem_g))
    for cp in fetches:
        cp.wait()
    for cc in range(DCHUNKS):
        pltpu.sync_copy(gbuf.at[:, pl.ds(cc * LANES, LANES)],
                        emb_s.at[cc, pl.ds(tok0, ROWS_PER_SUB)])
    plsc.subcore_barrier()

    pltpu.sync_copy(emb_s.at[cchunk], emb_v)

    def span_fetch(t):
        return pltpu.async_copy(
            span_hbm.at[pl.ds((node0 + t * CHUNK) * SPAN, CHUNK * SPAN)],
            span_v.at[t % 2], sem_span)

    span_cp = [span_fetch(0)]
    out_cp = [None, None]
    for t in range(NCHUNKS):
        par = t % 2
        span_cp[0].wait()
        if t + 1 < NCHUNKS:
            span_cp[0] = span_fetch(t + 1)
        if out_cp[par] is not None:
            out_cp[par].wait()

        def pair_body(k2, carry2):
            sv = span_v[par, pl.ds(k2 * 2 * SPAN, 2 * SPAN)]
            scale = jnp.float32(1.0 / SPAN)
            for half in range(2):
                rows = [emb_v[sv[half * SPAN + j]] for j in range(SPAN)]
                while len(rows) > 1:
                    rows = [rows[i] + rows[i + 1]
                            for i in range(0, len(rows), 2)]
                out_v[par, k2 * 2 + half] = rows[0] * scale
            return carry2

        lax.fori_loop(0, CHUNK // 2, pair_body, 0, unroll=5)
        out_cp[par] = pltpu.async_copy(
            out_v.at[par],
            out_hbm.at[pl.ds(node0 + t * CHUNK, CHUNK),
                       pl.ds(cchunk * LANES, LANES)],
            sem_out)
    for cp in out_cp:
        if cp is not None:
            cp.wait()


@jax.jit
def _graph_embed(tok, span, table1d):
    mesh = plsc.VectorSubcoreMesh(core_axis_name="c", subcore_axis_name="s",
                                  num_cores=NC, num_subcores=NS)
    f = pl.kernel(
        _body,
        out_type=jax.ShapeDtypeStruct((N_NODES, D), jnp.float32),
        mesh=mesh,
        scratch_types=[
            pltpu.VMEM((SEQ,), jnp.int32),
            pltpu.VMEM((ROWS_PER_SUB, D), jnp.float32),
            pltpu.VMEM((SEQ, LANES), jnp.float32),
            pltpu.VMEM((2, CHUNK * SPAN), jnp.int32),
            pltpu.VMEM((2, CHUNK, LANES), jnp.float32),
            pltpu.VMEM_SHARED((DCHUNKS, SEQ, LANES),
                              jnp.float32),
            pltpu.SemaphoreType.DMA,
            pltpu.SemaphoreType.DMA,
            pltpu.SemaphoreType.DMA,
        ],
        compiler_params=pltpu.CompilerParams(use_tc_tiling_on_sc=False,
                                             needs_layout_passes=False),
    )
    return f(tok, span, table1d)


def kernel(token_ids, node_span_idx, table):
    tok = token_ids.reshape(-1).astype(jnp.int32)
    span = node_span_idx.reshape(-1).astype(jnp.int32)
    table1d = table.reshape(-1)
    return _graph_embed(tok, span, table1d)


# speedup vs baseline: 6.7887x; 1.0225x over previous
import jax
import jax.numpy as jnp
from jax import lax
from jax.experimental import pallas as pl
from jax.experimental.pallas import tpu as pltpu
from jax.experimental.pallas import tpu_sc as plsc

VOCAB = 1000000
D = 64
SEQ = 4096
N_NODES = 50000
SPAN = 8

NC = 2
NS = 16
LANES = 16

DCHUNKS = D // LANES
NGROUPS = (NC * NS) // DCHUNKS
NODES_PER_GROUP = N_NODES // NGROUPS
CHUNK = 250
NCHUNKS = NODES_PER_GROUP // CHUNK
ROWS_PER_SUB = SEQ // NS


def _body(tok_hbm, span_hbm, table_hbm, out_hbm,
          tok_v, gbuf, emb_v, span_v, out_v,
          emb_s, sem_g, sem_span, sem_out):
    core = lax.axis_index("c")
    sub = lax.axis_index("s")
    group = core * (NS // DCHUNKS) + sub // DCHUNKS
    cchunk = sub % DCHUNKS
    tok0 = sub * ROWS_PER_SUB
    node0 = group * NODES_PER_GROUP

    pltpu.sync_copy(tok_hbm, tok_v)
    fetches = []
    for s in range(ROWS_PER_SUB // LANES):
        tv = tok_v[pl.ds(tok0 + s * LANES, LANES)]
        for l in range(LANES):
            fetches.append(pltpu.async_copy(
                table_hbm.at[pl.ds(tv[l] * D, D)],
                gbuf.at[s * LANES + l],
                sem_g))
    for cp in fetches:
        cp.wait()
    for cc in range(DCHUNKS):
        pltpu.sync_copy(gbuf.at[:, pl.ds(cc * LANES, LANES)],
                        emb_s.at[cc, pl.ds(tok0, ROWS_PER_SUB)])
    plsc.subcore_barrier()

    pltpu.sync_copy(emb_s.at[cchunk], emb_v)

    def span_fetch(t):
        return pltpu.async_copy(
            span_hbm.at[pl.ds((node0 + t * CHUNK) * SPAN, CHUNK * SPAN)],
            span_v.at[t % 2], sem_span)

    span_cp = [span_fetch(0)]
    out_cp = [None, None]
    for t in range(NCHUNKS):
        par = t % 2
        span_cp[0].wait()
        if t + 1 < NCHUNKS:
            span_cp[0] = span_fetch(t + 1)
        if out_cp[par] is not None:
            out_cp[par].wait()

        iota = lax.iota(jnp.int32, LANES)

        def pair_body(k2, carry2):
            sv = span_v[par, pl.ds(k2 * 2 * SPAN, 2 * SPAN)]
            scale = jnp.float32(1.0 / SPAN)
            for half in range(2):
                rows = [plsc.load_gather(
                            emb_v,
                            [jnp.full((LANES,), sv[half * SPAN + j]), iota])
                        for j in range(SPAN)]
                while len(rows) > 1:
                    rows = [rows[i] + rows[i + 1]
                            for i in range(0, len(rows), 2)]
                out_v[par, k2 * 2 + half] = rows[0] * scale
            return carry2

        lax.fori_loop(0, CHUNK // 2, pair_body, 0, unroll=5)
        out_cp[par] = pltpu.async_copy(
            out_v.at[par],
            out_hbm.at[pl.ds(node0 + t * CHUNK, CHUNK),
                       pl.ds(cchunk * LANES, LANES)],
            sem_out)
    for cp in out_cp:
        if cp is not None:
            cp.wait()


@jax.jit
def _graph_embed(tok, span, table1d):
    mesh = plsc.VectorSubcoreMesh(core_axis_name="c", subcore_axis_name="s",
                                  num_cores=NC, num_subcores=NS)
    f = pl.kernel(
        _body,
        out_type=jax.ShapeDtypeStruct((N_NODES, D), jnp.float32),
        mesh=mesh,
        scratch_types=[
            pltpu.VMEM((SEQ,), jnp.int32),
            pltpu.VMEM((ROWS_PER_SUB, D), jnp.float32),
            pltpu.VMEM((SEQ, LANES), jnp.float32),
            pltpu.VMEM((2, CHUNK * SPAN), jnp.int32),
            pltpu.VMEM((2, CHUNK, LANES), jnp.float32),
            pltpu.VMEM_SHARED((DCHUNKS, SEQ, LANES),
                              jnp.float32),
            pltpu.SemaphoreType.DMA,
            pltpu.SemaphoreType.DMA,
            pltpu.SemaphoreType.DMA,
        ],
        compiler_params=pltpu.CompilerParams(use_tc_tiling_on_sc=False,
                                             needs_layout_passes=False),
    )
    return f(tok, span, table1d)


def kernel(token_ids, node_span_idx, table):
    tok = token_ids.reshape(-1).astype(jnp.int32)
    span = node_span_idx.reshape(-1).astype(jnp.int32)
    table1d = table.reshape(-1)
    return _graph_embed(tok, span, table1d)


# speedup vs baseline: 7.4075x; 1.0911x over previous
import jax
import jax.numpy as jnp
from jax import lax
from jax.experimental import pallas as pl
from jax.experimental.pallas import tpu as pltpu
from jax.experimental.pallas import tpu_sc as plsc

VOCAB = 1000000
D = 64
SEQ = 4096
N_NODES = 50000
SPAN = 8

NC = 2
NS = 16
LANES = 16

DCHUNKS = D // LANES
NGROUPS = (NC * NS) // DCHUNKS
NODES_PER_GROUP = N_NODES // NGROUPS
CHUNK = 250
NCHUNKS = NODES_PER_GROUP // CHUNK
ROWS_PER_SUB = SEQ // NS


def _body(tok_hbm, span_hbm, table_hbm, out_hbm,
          tok_v, gbuf, emb_v, span_v, out_v,
          emb_s, sem_g, sem_span, sem_out):
    core = lax.axis_index("c")
    sub = lax.axis_index("s")
    group = core * (NS // DCHUNKS) + sub // DCHUNKS
    cchunk = sub % DCHUNKS
    tok0 = sub * ROWS_PER_SUB
    node0 = group * NODES_PER_GROUP

    pltpu.sync_copy(tok_hbm, tok_v)
    fetches = []
    for s in range(ROWS_PER_SUB // LANES):
        tv = tok_v[pl.ds(tok0 + s * LANES, LANES)]
        for l in range(LANES):
            fetches.append(pltpu.async_copy(
                table_hbm.at[tv[l]],
                gbuf.at[s * LANES + l],
                sem_g))
    for cp in fetches:
        cp.wait()
    for cc in range(DCHUNKS):
        pltpu.sync_copy(gbuf.at[:, pl.ds(cc * LANES, LANES)],
                        emb_s.at[cc, pl.ds(tok0, ROWS_PER_SUB)])
    plsc.subcore_barrier()

    pltpu.sync_copy(emb_s.at[cchunk], emb_v)

    def span_fetch(t):
        return pltpu.async_copy(
            span_hbm.at[pl.ds((node0 + t * CHUNK) * SPAN, CHUNK * SPAN)],
            span_v.at[t % 2], sem_span)

    span_cp = [span_fetch(0)]
    out_cp = [None, None]
    for t in range(NCHUNKS):
        par = t % 2
        span_cp[0].wait()
        if t + 1 < NCHUNKS:
            span_cp[0] = span_fetch(t + 1)
        if out_cp[par] is not None:
            out_cp[par].wait()

        iota = lax.iota(jnp.int32, LANES)

        def pair_body(k2, carry2):
            sv = span_v[par, pl.ds(k2 * 2 * SPAN, 2 * SPAN)]
            scale = jnp.float32(1.0 / SPAN)
            for half in range(2):
                rows = [plsc.load_gather(
                            emb_v,
                            [jnp.full((LANES,), sv[half * SPAN + j]), iota])
                        for j in range(SPAN)]
                while len(rows) > 1:
                    rows = [rows[i] + rows[i + 1]
                            for i in range(0, len(rows), 2)]
                out_v[par, k2 * 2 + half] = rows[0] * scale
            return carry2

        lax.fori_loop(0, CHUNK // 2, pair_body, 0, unroll=5)
        out_cp[par] = pltpu.async_copy(
            out_v.at[par],
            out_hbm.at[pl.ds(node0 + t * CHUNK, CHUNK),
                       pl.ds(cchunk * LANES, LANES)],
            sem_out)
    for cp in out_cp:
        if cp is not None:
            cp.wait()


@jax.jit
def _graph_embed(tok, span, table2d):
    mesh = plsc.VectorSubcoreMesh(core_axis_name="c", subcore_axis_name="s",
                                  num_cores=NC, num_subcores=NS)
    f = pl.kernel(
        _body,
        out_type=jax.ShapeDtypeStruct((N_NODES, D), jnp.float32),
        mesh=mesh,
        scratch_types=[
            pltpu.VMEM((SEQ,), jnp.int32),
            pltpu.VMEM((ROWS_PER_SUB, 2 * D), jnp.float32),
            pltpu.VMEM((SEQ, LANES), jnp.float32),
            pltpu.VMEM((2, CHUNK * SPAN), jnp.int32),
            pltpu.VMEM((2, CHUNK, LANES), jnp.float32),
            pltpu.VMEM_SHARED((DCHUNKS, SEQ, LANES),
                              jnp.float32),
            pltpu.SemaphoreType.DMA,
            pltpu.SemaphoreType.DMA,
            pltpu.SemaphoreType.DMA,
        ],
        compiler_params=pltpu.CompilerParams(use_tc_tiling_on_sc=False,
                                             needs_layout_passes=False),
    )
    return f(tok, span, table2d)


def kernel(token_ids, node_span_idx, table):
    tok = token_ids.reshape(-1).astype(jnp.int32)
    span = node_span_idx.reshape(-1).astype(jnp.int32)
    tablep = jnp.pad(table, ((0, 0), (0, D)))
    return _graph_embed(tok, span, tablep)


# speedup vs baseline: 7.6058x; 1.0268x over previous
import jax
import jax.numpy as jnp
from jax import lax
from jax.experimental import pallas as pl
from jax.experimental.pallas import tpu as pltpu
from jax.experimental.pallas import tpu_sc as plsc

VOCAB = 1000000
D = 64
SEQ = 4096
N_NODES = 50000
SPAN = 8

NC = 2
NS = 16
LANES = 16

DCHUNKS = D // LANES
NGROUPS = (NC * NS) // DCHUNKS
NODES_PER_GROUP = N_NODES // NGROUPS
CHUNK = 250
NCHUNKS = NODES_PER_GROUP // CHUNK
ROWS_PER_SUB = SEQ // NS


def _body(tok_hbm, span_hbm, table_hbm, out_hbm,
          tok_v, gbuf, emb_v, span_v, out_v,
          emb_s, sem_g, sem_span, sem_out):
    core = lax.axis_index("c")
    sub = lax.axis_index("s")
    group = core * (NS // DCHUNKS) + sub // DCHUNKS
    cchunk = sub % DCHUNKS
    tok0 = sub * ROWS_PER_SUB
    node0 = group * NODES_PER_GROUP

    pltpu.sync_copy(tok_hbm, tok_v)
    fetches = []
    for s in range(ROWS_PER_SUB // LANES):
        tv = tok_v[pl.ds(tok0 + s * LANES, LANES)]
        for l in range(LANES):
            fetches.append(pltpu.async_copy(
                table_hbm.at[tv[l]],
                gbuf.at[s * LANES + l],
                sem_g))
    for cp in fetches:
        cp.wait()
    for cc in range(DCHUNKS):
        pltpu.sync_copy(gbuf.at[:, pl.ds(cc * LANES, LANES)],
                        emb_s.at[cc, pl.ds(tok0, ROWS_PER_SUB)])
    plsc.subcore_barrier()

    pltpu.sync_copy(emb_s.at[cchunk], emb_v)

    def span_fetch(t):
        return pltpu.async_copy(
            span_hbm.at[pl.ds((node0 + t * CHUNK) * SPAN, CHUNK * SPAN)],
            span_v.at[t % 2], sem_span)

    span_cp = [span_fetch(0)]
    out_cp = [None, None]
    for t in range(NCHUNKS):
        par = t % 2
        span_cp[0].wait()
        if t + 1 < NCHUNKS:
            span_cp[0] = span_fetch(t + 1)
        if out_cp[par] is not None:
            out_cp[par].wait()

        iota = lax.iota(jnp.int32, LANES)

        def pair_body(k2, carry2):
            sv = span_v[par, pl.ds(k2 * 2 * SPAN, 2 * SPAN)]
            scale = jnp.float32(1.0 / SPAN)
            for half in range(2):
                rows = [plsc.load_gather(
                            emb_v,
                            [jnp.full((LANES,), sv[half * SPAN + j]), iota])
                        for j in range(SPAN)]
                while len(rows) > 1:
                    rows = [rows[i] + rows[i + 1]
                            for i in range(0, len(rows), 2)]
                out_v[par, k2 * 2 + half] = rows[0] * scale
            return carry2

        lax.fori_loop(0, CHUNK // 2, pair_body, 0, unroll=5)
        out_cp[par] = pltpu.async_copy(
            out_v.at[par],
            out_hbm.at[pl.ds(node0 + t * CHUNK, CHUNK),
                       pl.ds(cchunk * LANES, LANES)],
            sem_out)
    for cp in out_cp:
        if cp is not None:
            cp.wait()


@jax.jit
def _graph_embed(tok, span, table2d):
    mesh = plsc.VectorSubcoreMesh(core_axis_name="c", subcore_axis_name="s",
                                  num_cores=NC, num_subcores=NS)
    f = pl.kernel(
        _body,
        out_type=jax.ShapeDtypeStruct((N_NODES, 2 * D), jnp.float32),
        mesh=mesh,
        scratch_types=[
            pltpu.VMEM((SEQ,), jnp.int32),
            pltpu.VMEM((ROWS_PER_SUB, 2 * D), jnp.float32),
            pltpu.VMEM((SEQ, LANES), jnp.float32),
            pltpu.VMEM((2, CHUNK * SPAN), jnp.int32),
            pltpu.VMEM((2, CHUNK, LANES), jnp.float32),
            pltpu.VMEM_SHARED((DCHUNKS, SEQ, LANES),
                              jnp.float32),
            pltpu.SemaphoreType.DMA,
            pltpu.SemaphoreType.DMA,
            pltpu.SemaphoreType.DMA,
        ],
        compiler_params=pltpu.CompilerParams(use_tc_tiling_on_sc=False,
                                             needs_layout_passes=False),
    )
    return f(tok, span, table2d)


def kernel(token_ids, node_span_idx, table):
    tok = token_ids.reshape(-1).astype(jnp.int32)
    span = node_span_idx.reshape(-1).astype(jnp.int32)
    tablep = jnp.pad(table, ((0, 0), (0, D)))
    return _graph_embed(tok, span, tablep)[:, :D]
